# baseline (speedup 1.0000x reference)
import functools

import jax
import jax.numpy as jnp
from jax import lax
from jax.experimental import pallas as pl
from jax.experimental.pallas import tpu as pltpu
from jax.experimental.pallas import tpu_sc as plsc

VOCAB = 1000000
D = 64
B_TOTAL = 16384 * 50
NC, NS = 2, 16
NW = NC * NS
PER_W = B_TOTAL // NW
GRP = 128
G = PER_W // GRP
K = 2
SETW = K * GRP
C = G // K

_mesh = plsc.VectorSubcoreMesh(core_axis_name="c", subcore_axis_name="s")


@functools.partial(
    pl.kernel,
    out_type=jax.ShapeDtypeStruct((VOCAB, D), jnp.float32),
    mesh=_mesh,
    scratch_types=[
        pltpu.VMEM((G, GRP), jnp.int32),
        pltpu.VMEM((2, SETW, D), jnp.float32),
        pltpu.VMEM((2, SETW, D), jnp.float32),
        pltpu.SemaphoreType.DMA,
        pltpu.SemaphoreType.DMA,
        pltpu.SemaphoreType.DMA,
        pltpu.SemaphoreType.DMA,
    ],
    compiler_params=pltpu.CompilerParams(use_tc_tiling_on_sc=False),
)
def _embed(idx_hbm, table_hbm, out_hbm, idx_v, grows, srows,
           gsem0, gsem1, ssem0, ssem1):
    wid = lax.axis_index("s") * NC + lax.axis_index("c")
    gbase = wid * G
    gsems = (gsem0, gsem1)
    ssems = (ssem0, ssem1)

    pltpu.sync_copy(idx_hbm.at[pl.ds(gbase, G)], idx_v)

    def fire_gathers(c, s):
        for b in range(K):
            pltpu.make_async_copy(
                table_hbm.at[idx_v.at[c * K + b]],
                grows.at[s, pl.ds(b * GRP, GRP)], gsems[s]).start()

    def fire_scatters(c, s):
        for b in range(K):
            pltpu.make_async_copy(
                srows.at[s, pl.ds(b * GRP, GRP)],
                out_hbm.at[idx_v.at[c * K + b]], ssems[s]).start()

    def wait_set(buf, sems, s):
        pltpu.make_async_copy(
            out_hbm.at[pl.ds(0, SETW)], buf.at[s], sems[s]).wait()

    for s in range(2):
        fire_gathers(s, s)
        fire_scatters(s, s)

    def body(cc, carry):
        c0 = 2 * cc
        for s in range(2):
            wait_set(grows, gsems, s)
            fire_gathers(c0 + s, s)
            wait_set(srows, ssems, s)
            fire_scatters(c0 + s, s)
        return carry

    lax.fori_loop(1, C // 2, body, 0)
    for s in range(2):
        wait_set(grows, gsems, s)
        wait_set(srows, ssems, s)


def kernel(x, W):
    idx = x.reshape(B_TOTAL // GRP, GRP).astype(jnp.int32)
    out = _embed(idx, W)
    return out


# speedup vs baseline: 1.0627x; 1.0627x over previous
import functools

import jax
import jax.numpy as jnp
from jax import lax
from jax.experimental import pallas as pl
from jax.experimental.pallas import tpu as pltpu
from jax.experimental.pallas import tpu_sc as plsc

VOCAB = 1000000
D = 64
B_TOTAL = 16384 * 50
NC, NS = 2, 16
NW = NC * NS
PER_W = B_TOTAL // NW
GRP = 128
G = PER_W // GRP
K = 4
SETW = K * GRP
C = G // K

_mesh = plsc.VectorSubcoreMesh(core_axis_name="c", subcore_axis_name="s")


@functools.partial(
    pl.kernel,
    out_type=jax.ShapeDtypeStruct((B_TOTAL, D), jnp.float32),
    mesh=_mesh,
    scratch_types=[
        pltpu.VMEM((G, GRP), jnp.int32),
        pltpu.VMEM((2, SETW, D), jnp.float32),
        pltpu.SemaphoreType.DMA,
        pltpu.SemaphoreType.DMA,
        pltpu.SemaphoreType.DMA,
        pltpu.SemaphoreType.DMA,
    ],
    compiler_params=pltpu.CompilerParams(use_tc_tiling_on_sc=False),
)
def _embed(idx_hbm, table_hbm, out_hbm, idx_v, rows, gsem0, gsem1, osem0, osem1):
    wid = lax.axis_index("s") * NC + lax.axis_index("c")
    gbase = wid * G
    obase = wid * PER_W
    gsems = (gsem0, gsem1)
    osems = (osem0, osem1)

    pltpu.sync_copy(idx_hbm.at[pl.ds(gbase, G)], idx_v)

    def fire_gathers(c, s):
        for b in range(K):
            pltpu.make_async_copy(
                table_hbm.at[idx_v.at[c * K + b]],
                rows.at[s, pl.ds(b * GRP, GRP)], gsems[s]).start()

    def wait_gathers(s):
        pltpu.make_async_copy(
            out_hbm.at[pl.ds(0, SETW)], rows.at[s], gsems[s]).wait()

    def out_copy(c, s):
        return pltpu.make_async_copy(
            rows.at[s], out_hbm.at[pl.ds(obase + c * SETW, SETW)], osems[s])

    fire_gathers(0, 0)
    fire_gathers(1, 1)

    def super_body(cc, carry):
        c0 = 2 * cc
        for s in range(2):
            wait_gathers(s)
            out_copy(c0 + s, s).start()
        for s in range(2):
            out_copy(c0 + s, s).wait()
            fire_gathers(c0 + 2 + s, s)
        return carry

    lax.fori_loop(0, C // 2 - 1, super_body, 0)
    cL = C - 2
    for s in range(2):
        wait_gathers(s)
        out_copy(cL + s, s).start()
    for s in range(2):
        out_copy(cL + s, s).wait()


def kernel(x, W):
    idx = x.reshape(B_TOTAL // GRP, GRP).astype(jnp.int32)
    out = _embed(idx, W)
    return out.reshape(x.shape[0], x.shape[1], D)


# speedup vs baseline: 1.0657x; 1.0028x over previous
import functools

import jax
import jax.numpy as jnp
from jax import lax
from jax.experimental import pallas as pl
from jax.experimental.pallas import tpu as pltpu
from jax.experimental.pallas import tpu_sc as plsc

VOCAB = 1000000
D = 64
B_TOTAL = 16384 * 50
NC, NS = 2, 16
NW = NC * NS
PER_W = B_TOTAL // NW
GRP = 128
G = PER_W // GRP
K = 2
NSET = 4
SETW = K * GRP
C = G // K

_mesh = plsc.VectorSubcoreMesh(core_axis_name="c", subcore_axis_name="s")


@functools.partial(
    pl.kernel,
    out_type=jax.ShapeDtypeStruct((B_TOTAL, D), jnp.float32),
    mesh=_mesh,
    scratch_types=[
        pltpu.VMEM((G, GRP), jnp.int32),
        pltpu.VMEM((NSET, SETW, D), jnp.float32),
        pltpu.SemaphoreType.DMA,
        pltpu.SemaphoreType.DMA,
        pltpu.SemaphoreType.DMA,
        pltpu.SemaphoreType.DMA,
        pltpu.SemaphoreType.DMA,
        pltpu.SemaphoreType.DMA,
        pltpu.SemaphoreType.DMA,
        pltpu.SemaphoreType.DMA,
    ],
    compiler_params=pltpu.CompilerParams(use_tc_tiling_on_sc=False),
)
def _embed(idx_hbm, table_hbm, out_hbm, idx_v, rows,
           gsem0, gsem1, gsem2, gsem3, osem0, osem1, osem2, osem3):
    wid = lax.axis_index("s") * NC + lax.axis_index("c")
    gbase = wid * G
    obase = wid * PER_W
    gsems = (gsem0, gsem1, gsem2, gsem3)
    osems = (osem0, osem1, osem2, osem3)

    pltpu.sync_copy(idx_hbm.at[pl.ds(gbase, G)], idx_v)

    def fire_gathers(c, s):
        for b in range(K):
            pltpu.make_async_copy(
                table_hbm.at[idx_v.at[c * K + b]],
                rows.at[s, pl.ds(b * GRP, GRP)], gsems[s]).start()

    def wait_gathers(s):
        pltpu.make_async_copy(
            out_hbm.at[pl.ds(0, SETW)], rows.at[s], gsems[s]).wait()

    def out_copy(c, s):
        return pltpu.make_async_copy(
            rows.at[s], out_hbm.at[pl.ds(obase + c * SETW, SETW)], osems[s])

    for s in range(NSET):
        fire_gathers(s, s)

    def super_body(cc, carry):
        c0 = NSET * cc
        for s in range(NSET):
            wait_gathers(s)
            out_copy(c0 + s, s).start()
        for s in range(NSET):
            out_copy(c0 + s, s).wait()
            fire_gathers(c0 + NSET + s, s)
        return carry

    lax.fori_loop(0, C // NSET - 1, super_body, 0)
    cL = C - NSET
    for s in range(NSET):
        wait_gathers(s)
        out_copy(cL + s, s).start()
    for s in range(NSET):
        out_copy(cL + s, s).wait()


def kernel(x, W):
    idx = x.reshape(B_TOTAL // GRP, GRP).astype(jnp.int32)
    out = _embed(idx, W)
    return out.reshape(x.shape[0], x.shape[1], D)


# speedup vs baseline: 1.0658x; 1.0001x over previous
import functools

import jax
import jax.numpy as jnp
from jax import lax
from jax.experimental import pallas as pl
from jax.experimental.pallas import tpu as pltpu
from jax.experimental.pallas import tpu_sc as plsc

VOCAB = 1000000
D = 64
B_TOTAL = 16384 * 50
NC, NS = 2, 16
NW = NC * NS
PER_W = B_TOTAL // NW
GRP = 256
G = PER_W // GRP
K = 1
NSET = 4
SETW = K * GRP
C = G // K

_mesh = plsc.VectorSubcoreMesh(core_axis_name="c", subcore_axis_name="s")


@functools.partial(
    pl.kernel,
    out_type=jax.ShapeDtypeStruct((B_TOTAL, D), jnp.float32),
    mesh=_mesh,
    scratch_types=[
        pltpu.VMEM((G, GRP), jnp.int32),
        pltpu.VMEM((NSET, SETW, D), jnp.float32),
        pltpu.SemaphoreType.DMA,
        pltpu.SemaphoreType.DMA,
        pltpu.SemaphoreType.DMA,
        pltpu.SemaphoreType.DMA,
        pltpu.SemaphoreType.DMA,
        pltpu.SemaphoreType.DMA,
        pltpu.SemaphoreType.DMA,
        pltpu.SemaphoreType.DMA,
    ],
    compiler_params=pltpu.CompilerParams(use_tc_tiling_on_sc=False),
)
def _embed(idx_hbm, table_hbm, out_hbm, idx_v, rows,
           gsem0, gsem1, gsem2, gsem3, osem0, osem1, osem2, osem3):
    wid = lax.axis_index("s") * NC + lax.axis_index("c")
    gbase = wid * G
    obase = wid * PER_W
    gsems = (gsem0, gsem1, gsem2, gsem3)
    osems = (osem0, osem1, osem2, osem3)

    pltpu.sync_copy(idx_hbm.at[pl.ds(gbase, G)], idx_v)

    def fire_gathers(c, s):
        for b in range(K):
            pltpu.make_async_copy(
                table_hbm.at[idx_v.at[c * K + b]],
                rows.at[s, pl.ds(b * GRP, GRP)], gsems[s]).start()

    def wait_gathers(s):
        pltpu.make_async_copy(
            out_hbm.at[pl.ds(0, SETW)], rows.at[s], gsems[s]).wait()

    def out_copy(c, s):
        return pltpu.make_async_copy(
            rows.at[s], out_hbm.at[pl.ds(obase + c * SETW, SETW)], osems[s])

    for s in range(NSET):
        fire_gathers(s, s)

    def super_body(cc, carry):
        c0 = NSET * cc
        for s in range(NSET):
            wait_gathers(s)
            out_copy(c0 + s, s).start()
        for s in range(NSET):
            out_copy(c0 + s, s).wait()
            fire_gathers(c0 + NSET + s, s)
        return carry

    lax.fori_loop(0, C // NSET - 1, super_body, 0)
    cL = C - NSET
    for s in range(NSET):
        wait_gathers(s)
        out_copy(cL + s, s).start()
    for s in range(NSET):
        out_copy(cL + s, s).wait()


def kernel(x, W):
    idx = x.reshape(B_TOTAL // GRP, GRP).astype(jnp.int32)
    out = _embed(idx, W)
    return out.reshape(x.shape[0], x.shape[1], D)
